# baseline (speedup 1.0000x reference)
import jax
import jax.numpy as jnp
from jax.experimental import pallas as pl
from jax.experimental.pallas import tpu as pltpu

_K = 25
_PAD = (_K - 1) // 2
_FRONT = 16


def _decomp_body(x_ref, res_ref, mm_ref, xp_ref):
    bb = x_ref.shape[0]
    t, c = x_ref.shape[1], x_ref.shape[2]
    span = t + 2 * _FRONT
    for n in range(bb):
        x = x_ref[n]
        base = n * span
        xp_ref[base:base + _FRONT] = jnp.broadcast_to(x[0:1], (_FRONT, c))
        xp_ref[base + _FRONT:base + _FRONT + t] = x
        xp_ref[base + _FRONT + t:base + span] = jnp.broadcast_to(
            x[t - 1:t], (_FRONT, c))
    y = xp_ref[...]
    m = y.shape[0]
    c2 = y[:-1] + y[1:]
    c4 = c2[:-2] + c2[2:]
    c8 = c4[:-4] + c4[4:]
    c16 = c8[:-8] + c8[8:]
    c24 = c16[0:m - 24] + c8[16:m - 8]
    c25 = c24 + y[24:m]
    for n in range(bb):
        base = n * span
        mm = c25[base + 4:base + 4 + t] * (1.0 / _K)
        res_ref[n] = y[base + _FRONT:base + _FRONT + t] - mm
        mm_ref[n] = mm


def kernel(x):
    b, t, c = x.shape
    out = jax.ShapeDtypeStruct((b, t, c), x.dtype)
    bb = 2
    grid = (b // bb,)
    spec = pl.BlockSpec((bb, t, c), lambda i: (i, 0, 0))
    res, mm = pl.pallas_call(
        _decomp_body,
        grid=grid,
        in_specs=[spec],
        out_specs=(spec, spec),
        out_shape=(out, out),
        scratch_shapes=[pltpu.VMEM((bb * (t + 2 * _FRONT), c), x.dtype)],
    )(x)
    return (res, mm)


# speedup vs baseline: 1.0031x; 1.0031x over previous
import jax
import jax.numpy as jnp
from jax.experimental import pallas as pl
from jax.experimental.pallas import tpu as pltpu

_K = 25
_PAD = (_K - 1) // 2
_FRONT = 16


def _decomp_body(x_ref, res_ref, mm_ref, xp_ref):
    for n in range(x_ref.shape[0]):
        _decomp_one(n, x_ref, res_ref, mm_ref, xp_ref)


def _decomp_one(n, x_ref, res_ref, mm_ref, xp_ref):
    x = x_ref[n]
    t, c = x.shape
    xp_ref[0:_FRONT] = jnp.broadcast_to(x[0:1], (_FRONT, c))
    xp_ref[_FRONT:_FRONT + t] = x
    xp_ref[_FRONT + t:] = jnp.broadcast_to(x[t - 1:t], (_FRONT, c))
    y = xp_ref[...]
    c2 = y[:-1] + y[1:]
    c4 = c2[:-2] + c2[2:]
    c8 = c4[:-4] + c4[4:]
    c16 = c8[:-8] + c8[8:]
    c24 = c16[0:t + 8] + c8[16:16 + t + 8]
    c25 = c24 + y[24:24 + t + 8]
    mm = c25[4:4 + t] * (1.0 / _K)
    res_ref[n] = x - mm
    mm_ref[n] = mm


def kernel(x):
    b, t, c = x.shape
    out = jax.ShapeDtypeStruct((b, t, c), x.dtype)
    bb = 2
    grid = (b // bb,)
    spec = pl.BlockSpec((bb, t, c), lambda i: (i, 0, 0))
    res, mm = pl.pallas_call(
        _decomp_body,
        grid=grid,
        in_specs=[spec],
        out_specs=(spec, spec),
        out_shape=(out, out),
        scratch_shapes=[pltpu.VMEM((t + 2 * _FRONT, c), x.dtype)],
    )(x)
    return (res, mm)
